# baseline (speedup 1.0000x reference)
import dataclasses
import functools

import jax
import jax.numpy as jnp
from jax import lax
from jax.experimental import pallas as pl
from jax.experimental.pallas import tpu as pltpu
from jax.experimental.pallas import tpu_sc as plsc

N = 10000
NP = 10240
E = 160000
EP = 163840
G = 64
F0 = 512
F1 = 384
F2 = 128
BN = 256
NBLK = NP // BN
RPT = NP // 16

_mesh = plsc.VectorSubcoreMesh(core_axis_name="c", subcore_axis_name="s")

_sc_params = pltpu.CompilerParams()
if "needs_layout_passes" in pltpu.CompilerParams.__dataclass_fields__:
    _sc_params = dataclasses.replace(_sc_params, needs_layout_passes=False)


@functools.partial(
    pl.kernel,
    out_type=jax.ShapeDtypeStruct((32, 80, 128), jnp.float32),
    mesh=_mesh,
    compiler_params=_sc_params,
    scratch_types=[
        pltpu.VMEM((EP // 32,), jnp.int32),
        pltpu.VMEM((80, 128), jnp.float32),
    ],
)
def _deg(dst_hbm, out_hbm, dst_v, deg_v):
    wid = lax.axis_index("s") * 2 + lax.axis_index("c")
    pltpu.sync_copy(dst_hbm.at[wid], dst_v)
    zero16 = jnp.zeros((16,), jnp.float32)

    @pl.loop(0, 80)
    def _zero_row(r):
        @pl.loop(0, 128, step=16)
        def _zero_col(cc):
            deg_v[r, pl.ds(cc, 16)] = zero16

    one16 = jnp.ones((16,), jnp.float32)

    @pl.loop(0, EP // 32, step=16)
    def _hist(i):
        d = dst_v[pl.ds(i, 16)]
        plsc.addupdate_scatter(
            deg_v, [jnp.right_shift(d, 7), jnp.bitwise_and(d, 127)], one16
        )

    pltpu.sync_copy(deg_v, out_hbm.at[wid])


@functools.partial(
    pl.kernel,
    out_type=jax.ShapeDtypeStruct((4 * NP, 128), jnp.float32),
    mesh=_mesh,
    compiler_params=_sc_params,
    scratch_types=[
        pltpu.VMEM((40, 64), jnp.int32),
        pltpu.VMEM((20, 128), jnp.int32),
        pltpu.VMEM((128, 128), jnp.float32),
        pltpu.VMEM((128, 128), jnp.float32),
        pltpu.VMEM_SHARED((NP, 128), jnp.float32),
        [pltpu.SemaphoreType.DMA] * 4,
    ],
)
def _agg(hn_hbm, src4_hbm, dst_hbm, zeros_hbm, out_hbm,
         src_v, dst_v, gA, gB, acc, gsems):
    cid = lax.axis_index("c")
    sid = lax.axis_index("s")

    @pl.loop(0, 2)
    def _chunk(k):
        chunk = cid * 2 + k
        pltpu.sync_copy(zeros_hbm, acc.at[pl.ds(sid * RPT, RPT)])
        plsc.subcore_barrier()

        @pl.loop(0, 4)
        def _quarter(q):
            pltpu.sync_copy(src4_hbm.at[(chunk * 16 + sid) * 4 + q], src_v)
            pltpu.sync_copy(dst_hbm.at[sid * 4 + q], dst_v)

            def gstart(j, buf, half, sem):
                pltpu.async_copy(hn_hbm.at[src_v.at[j]],
                                 buf.at[pl.ds(half * 64, 64)], sem)

            def gwait(j, buf, half, sem):
                pltpu.make_async_copy(hn_hbm.at[src_v.at[j]],
                                      buf.at[pl.ds(half * 64, 64)],
                                      sem).wait()

            def pair(p, buf, slo, shi, refill):
                gwait(2 * p, buf, 0, slo)
                gwait(2 * p + 1, buf, 1, shi)
                pltpu.sync_copy(buf, acc.at[dst_v.at[p]], add=True)
                if refill:
                    gstart(2 * p + 4, buf, 0, slo)
                    gstart(2 * p + 5, buf, 1, shi)

            gstart(0, gA, 0, gsems[0])
            gstart(1, gA, 1, gsems[1])
            gstart(2, gB, 0, gsems[2])
            gstart(3, gB, 1, gsems[3])

            @pl.loop(0, 9)
            def _pairs(t):
                pair(2 * t, gA, gsems[0], gsems[1], True)
                pair(2 * t + 1, gB, gsems[2], gsems[3], True)

            pair(18, gA, gsems[0], gsems[1], False)
            pair(19, gB, gsems[2], gsems[3], False)

        plsc.subcore_barrier()
        pltpu.sync_copy(acc.at[pl.ds(sid * RPT, RPT)],
                        out_hbm.at[pl.ds(chunk * NP + sid * RPT, RPT)])
        plsc.subcore_barrier()


def _mlp0_body(degp, xb, w0, b0, hn4o, dinvo):
    i = pl.program_id(0)
    deg = jnp.sum(degp[...], axis=0) + 1.0
    rows = i * BN + lax.broadcasted_iota(jnp.int32, (BN, 1), 0)
    dinv = jnp.where(rows < N, lax.rsqrt(deg), 0.0)
    h = jnp.dot(xb[...], w0[...], preferred_element_type=jnp.float32) + b0[...]
    h = jnp.maximum(h, 0.0) * dinv
    for c in range(4):
        hn4o[c] = h[:, c * 128:(c + 1) * 128]
    dinvo[...] = dinv


def _bf(x):
    return x.astype(jnp.bfloat16)


def _mlp0(degp, x_pad, w0t, b0p):
    return pl.pallas_call(
        _mlp0_body,
        grid=(NBLK,),
        in_specs=[
            pl.BlockSpec((32, BN, 1), lambda i: (0, i, 0)),
            pl.BlockSpec((BN, 128), lambda i: (i, 0)),
            pl.BlockSpec((128, F0), lambda i: (0, 0)),
            pl.BlockSpec((1, F0), lambda i: (0, 0)),
        ],
        out_specs=[pl.BlockSpec((4, BN, 128), lambda i: (0, i, 0)),
                   pl.BlockSpec((BN, 1), lambda i: (i, 0))],
        out_shape=[jax.ShapeDtypeStruct((4, NP, 128), jnp.float32),
                   jax.ShapeDtypeStruct((NP, 1), jnp.float32)],
    )(degp, x_pad, w0t, b0p)


def _mlp2_body(a4, h4, dinv, wc, bc,
               w1, b1, w2, b2, w3, b3, b2d, out, h4s):
    i = pl.program_id(0)
    d = dinv[...]
    acc = jnp.zeros((BN, F0), jnp.float32)
    for c in range(4):
        z = _bf((a4[c] + h4[c]) * d)
        acc += jnp.dot(z, wc[c * 128:(c + 1) * 128, :],
                       preferred_element_type=jnp.float32)
    x1 = _bf(jnp.maximum(acc + bc[...], 0.0))
    x2 = _bf(jnp.maximum(
        jnp.dot(x1, w1[...], preferred_element_type=jnp.float32) + b1[...],
        0.0))
    x3 = _bf(jnp.maximum(
        jnp.dot(x2, w2[...], preferred_element_type=jnp.float32) + b2[...],
        0.0))
    h4s[pl.ds(i * BN, BN), :] = (
        jnp.dot(x3, w3[...], preferred_element_type=jnp.float32) + b3[...])

    @pl.when(i == NBLK - 1)
    def _():
        b = b2d[...]
        neg = jnp.full((8, 128), -jnp.inf, jnp.float32)

        def seg(s, carry):
            start = jnp.sum((b < s).astype(jnp.int32))
            end = jnp.sum((b < s + 1).astype(jnp.int32))

            def grp(gi, m):
                rows = gi * 8 + lax.broadcasted_iota(jnp.int32, (8, 128), 0)
                v = h4s[pl.ds(gi * 8, 8), :]
                return jnp.maximum(
                    m, jnp.where((rows >= start) & (rows < end), v, -jnp.inf))

            m = lax.fori_loop(start // 8, (end + 7) // 8, grp, neg)
            out[pl.ds(s, 1), :] = jnp.max(m, axis=0, keepdims=True)
            return carry

        lax.fori_loop(0, G, seg, 0)
        g = out[...]
        mx = jnp.max(g, axis=0, keepdims=True)
        e = jnp.exp(g - mx)
        out[...] = e / jnp.sum(e, axis=0, keepdims=True)


def _mlp2(agg4, hn4, dinv, wct, bcp, w1t, b1p, w2t, b2p, w3t, b3p, batch2d):
    stk = pl.BlockSpec((4, BN, 128), lambda i: (0, i, 0))
    full = lambda r, c: pl.BlockSpec((r, c), lambda i: (0, 0))
    return pl.pallas_call(
        _mlp2_body,
        grid=(NBLK,),
        in_specs=[stk, stk] + [
            pl.BlockSpec((BN, 1), lambda i: (i, 0)),
            full(F0, F0), full(1, F0),
            full(F0, F1), full(1, F1),
            full(F1, F2), full(1, F2),
            full(F2, 128), full(1, 128),
            full(80, 128),
        ],
        out_specs=pl.BlockSpec((G, 128), lambda i: (0, 0)),
        out_shape=jax.ShapeDtypeStruct((G, 128), jnp.float32),
        scratch_shapes=[pltpu.VMEM((NP, 128), jnp.float32)],
    )(agg4, hn4, dinv, wct, bcp, w1t, b1p, w2t, b2p, w3t, b3p, batch2d)


def kernel(x, edge_index, batch, W0, b0, Wc, bc, W1, b1, W2, b2, W3, b3):
    f32 = jnp.float32
    x_pad = jnp.zeros((NP, 128), f32).at[:N, :19].set(x)
    batch2d = jnp.full((NP,), G, jnp.int32).at[:N].set(batch).reshape(80, 128)
    src = jnp.full((EP,), N, jnp.int32).at[:E].set(edge_index[0])
    dst = jnp.full((EP,), N, jnp.int32).at[:E].set(edge_index[1])
    src4 = (src[None, :] + (jnp.arange(4, dtype=jnp.int32) * NP)[:, None])
    src4 = src4.reshape(256, 40, 64)
    dst3 = dst.reshape(64, 20, 128)
    dst32 = dst.reshape(32, EP // 32)
    zrows = jnp.zeros((RPT, 128), f32)

    w0t = jnp.zeros((128, F0), f32).at[:19, :500].set(W0.T)
    b0p = jnp.zeros((1, F0), f32).at[0, :500].set(b0)
    wct = _bf(jnp.zeros((F0, F0), f32).at[:500, :500].set(Wc.T))
    bcp = jnp.zeros((1, F0), f32).at[0, :500].set(bc)
    w1t = _bf(jnp.zeros((F0, F1), f32).at[:500, :300].set(W1.T))
    b1p = jnp.zeros((1, F1), f32).at[0, :300].set(b1)
    w2t = _bf(jnp.zeros((F1, F2), f32).at[:300, :100].set(W2.T))
    b2p = jnp.zeros((1, F2), f32).at[0, :100].set(b2)
    w3t = _bf(jnp.zeros((F2, 128), f32).at[:100, :6].set(W3.T))
    b3p = jnp.zeros((1, 128), f32).at[0, :6].set(b3)

    degp = _deg(dst32)
    degp = degp.reshape(32, NP, 1)
    hn4, dinv = _mlp0(degp, x_pad, w0t, b0p)
    agg_all = _agg(hn4.reshape(4 * NP, 128), src4, dst3, zrows)
    gout = _mlp2(agg_all.reshape(4, NP, 128), hn4, dinv,
                 wct, bcp, w1t, b1p, w2t, b2p, w3t, b3p, batch2d)
    return gout[:, :6]


# speedup vs baseline: 1.0615x; 1.0615x over previous
import dataclasses
import functools

import jax
import jax.numpy as jnp
from jax import lax
from jax.experimental import pallas as pl
from jax.experimental.pallas import tpu as pltpu
from jax.experimental.pallas import tpu_sc as plsc

N = 10000
NP = 10240
E = 160000
EP = 163840
G = 64
F0 = 512
F1 = 384
F2 = 128
BN = 256
NBLK = NP // BN
RPT = NP // 16

_mesh = plsc.VectorSubcoreMesh(core_axis_name="c", subcore_axis_name="s")

_sc_params = pltpu.CompilerParams()
if "needs_layout_passes" in pltpu.CompilerParams.__dataclass_fields__:
    _sc_params = dataclasses.replace(_sc_params, needs_layout_passes=False)


@functools.partial(
    pl.kernel,
    out_type=jax.ShapeDtypeStruct((32, 80, 128), jnp.float32),
    mesh=_mesh,
    compiler_params=_sc_params,
    scratch_types=[
        pltpu.VMEM((EP // 32,), jnp.int32),
        pltpu.VMEM((80, 128), jnp.float32),
    ],
)
def _deg(dst_hbm, out_hbm, dst_v, deg_v):
    wid = lax.axis_index("s") * 2 + lax.axis_index("c")
    pltpu.sync_copy(dst_hbm.at[wid], dst_v)
    zero16 = jnp.zeros((16,), jnp.float32)

    @pl.loop(0, 80)
    def _zero_row(r):
        @pl.loop(0, 128, step=16)
        def _zero_col(cc):
            deg_v[r, pl.ds(cc, 16)] = zero16

    one16 = jnp.ones((16,), jnp.float32)

    @pl.loop(0, EP // 32, step=16)
    def _hist(i):
        d = dst_v[pl.ds(i, 16)]
        plsc.addupdate_scatter(
            deg_v, [jnp.right_shift(d, 7), jnp.bitwise_and(d, 127)], one16
        )

    pltpu.sync_copy(deg_v, out_hbm.at[wid])


@functools.partial(
    pl.kernel,
    out_type=jax.ShapeDtypeStruct((4 * NP, 128), jnp.float32),
    mesh=_mesh,
    compiler_params=_sc_params,
    scratch_types=[
        pltpu.VMEM((80, 64), jnp.int32),
        pltpu.VMEM((80, 64), jnp.int32),
        pltpu.VMEM((2, 64, 128), jnp.float32),
        pltpu.VMEM((64, 128), jnp.float32),
        pltpu.VMEM_SHARED((NP, 128), jnp.float32),
        [pltpu.SemaphoreType.DMA] * 3,
    ],
)
def _agg(hn_hbm, src4_hbm, dst_hbm, zeros_hbm, out_hbm,
         src_v, dst_v, gbuf01, gbuf2, acc, gsems):
    cid = lax.axis_index("c")
    sid = lax.axis_index("s")

    def buf(b):
        return gbuf2 if b == 2 else gbuf01.at[b]

    @pl.loop(0, 2)
    def _chunk(k):
        chunk = cid * 2 + k
        pltpu.sync_copy(zeros_hbm, acc.at[pl.ds(sid * RPT, RPT)])
        plsc.subcore_barrier()

        @pl.loop(0, 2)
        def _half(h):
            pltpu.sync_copy(src4_hbm.at[(chunk * 16 + sid) * 2 + h], src_v)
            pltpu.sync_copy(dst_hbm.at[sid * 2 + h], dst_v)

            def gstart(j, b):
                pltpu.async_copy(hn_hbm.at[src_v.at[j]], buf(b), gsems[b])

            def gwait(j, b):
                pltpu.make_async_copy(hn_hbm.at[src_v.at[j]], buf(b),
                                      gsems[b]).wait()

            def step(j, b, refill):
                gwait(j, b)
                pltpu.sync_copy(buf(b), acc.at[dst_v.at[j]], add=True)
                if refill:
                    gstart(j + 3, b)

            for b in range(3):
                gstart(b, b)

            @pl.loop(0, 25)
            def _edge_batches(t):
                for b in range(3):
                    step(t * 3 + b, b, True)

            step(75, 0, True)
            step(76, 1, True)
            step(77, 2, False)
            step(78, 0, False)
            step(79, 1, False)

        plsc.subcore_barrier()
        pltpu.sync_copy(acc.at[pl.ds(sid * RPT, RPT)],
                        out_hbm.at[pl.ds(chunk * NP + sid * RPT, RPT)])
        plsc.subcore_barrier()


def _mlp0_body(degp, xb, w0, b0, hn4o, dinvo):
    i = pl.program_id(0)
    deg = jnp.sum(degp[...], axis=0) + 1.0
    rows = i * BN + lax.broadcasted_iota(jnp.int32, (BN, 1), 0)
    dinv = jnp.where(rows < N, lax.rsqrt(deg), 0.0)
    h = jnp.dot(xb[...], w0[...], preferred_element_type=jnp.float32) + b0[...]
    h = jnp.maximum(h, 0.0) * dinv
    for c in range(4):
        hn4o[c] = h[:, c * 128:(c + 1) * 128]
    dinvo[...] = dinv


def _bf(x):
    return x.astype(jnp.bfloat16)


def _mlp0(degp, x_pad, w0t, b0p):
    return pl.pallas_call(
        _mlp0_body,
        grid=(NBLK,),
        in_specs=[
            pl.BlockSpec((32, BN, 1), lambda i: (0, i, 0)),
            pl.BlockSpec((BN, 128), lambda i: (i, 0)),
            pl.BlockSpec((128, F0), lambda i: (0, 0)),
            pl.BlockSpec((1, F0), lambda i: (0, 0)),
        ],
        out_specs=[pl.BlockSpec((4, BN, 128), lambda i: (0, i, 0)),
                   pl.BlockSpec((BN, 1), lambda i: (i, 0))],
        out_shape=[jax.ShapeDtypeStruct((4, NP, 128), jnp.float32),
                   jax.ShapeDtypeStruct((NP, 1), jnp.float32)],
    )(degp, x_pad, w0t, b0p)


def _mlp2_body(a4, h4, dinv, wc, bc,
               w1, b1, w2, b2, w3, b3, b2d, out, h4s):
    i = pl.program_id(0)
    d = dinv[...]
    acc = jnp.zeros((BN, F0), jnp.float32)
    for c in range(4):
        z = _bf((a4[c] + h4[c]) * d)
        acc += jnp.dot(z, wc[c * 128:(c + 1) * 128, :],
                       preferred_element_type=jnp.float32)
    x1 = _bf(jnp.maximum(acc + bc[...], 0.0))
    x2 = _bf(jnp.maximum(
        jnp.dot(x1, w1[...], preferred_element_type=jnp.float32) + b1[...],
        0.0))
    x3 = _bf(jnp.maximum(
        jnp.dot(x2, w2[...], preferred_element_type=jnp.float32) + b2[...],
        0.0))
    h4s[pl.ds(i * BN, BN), :] = (
        jnp.dot(x3, w3[...], preferred_element_type=jnp.float32) + b3[...])

    @pl.when(i == NBLK - 1)
    def _():
        b = b2d[...]
        neg = jnp.full((8, 128), -jnp.inf, jnp.float32)

        def seg(s, carry):
            start = jnp.sum((b < s).astype(jnp.int32))
            end = jnp.sum((b < s + 1).astype(jnp.int32))

            def grp(gi, m):
                rows = gi * 8 + lax.broadcasted_iota(jnp.int32, (8, 128), 0)
                v = h4s[pl.ds(gi * 8, 8), :]
                return jnp.maximum(
                    m, jnp.where((rows >= start) & (rows < end), v, -jnp.inf))

            m = lax.fori_loop(start // 8, (end + 7) // 8, grp, neg)
            out[pl.ds(s, 1), :] = jnp.max(m, axis=0, keepdims=True)
            return carry

        lax.fori_loop(0, G, seg, 0)
        g = out[...]
        mx = jnp.max(g, axis=0, keepdims=True)
        e = jnp.exp(g - mx)
        out[...] = e / jnp.sum(e, axis=0, keepdims=True)


def _mlp2(agg4, hn4, dinv, wct, bcp, w1t, b1p, w2t, b2p, w3t, b3p, batch2d):
    stk = pl.BlockSpec((4, BN, 128), lambda i: (0, i, 0))
    full = lambda r, c: pl.BlockSpec((r, c), lambda i: (0, 0))
    return pl.pallas_call(
        _mlp2_body,
        grid=(NBLK,),
        in_specs=[stk, stk] + [
            pl.BlockSpec((BN, 1), lambda i: (i, 0)),
            full(F0, F0), full(1, F0),
            full(F0, F1), full(1, F1),
            full(F1, F2), full(1, F2),
            full(F2, 128), full(1, 128),
            full(80, 128),
        ],
        out_specs=pl.BlockSpec((G, 128), lambda i: (0, 0)),
        out_shape=jax.ShapeDtypeStruct((G, 128), jnp.float32),
        scratch_shapes=[pltpu.VMEM((NP, 128), jnp.float32)],
    )(agg4, hn4, dinv, wct, bcp, w1t, b1p, w2t, b2p, w3t, b3p, batch2d)


def kernel(x, edge_index, batch, W0, b0, Wc, bc, W1, b1, W2, b2, W3, b3):
    f32 = jnp.float32
    x_pad = jnp.zeros((NP, 128), f32).at[:N, :19].set(x)
    batch2d = jnp.full((NP,), G, jnp.int32).at[:N].set(batch).reshape(80, 128)
    src = jnp.full((EP,), N, jnp.int32).at[:E].set(edge_index[0])
    dst = jnp.full((EP,), N, jnp.int32).at[:E].set(edge_index[1])
    src4 = (src[None, :] + (jnp.arange(4, dtype=jnp.int32) * NP)[:, None])
    src4 = src4.reshape(128, 80, 64)
    dst3 = dst.reshape(32, 80, 64)
    dst32 = dst.reshape(32, EP // 32)
    zrows = jnp.zeros((RPT, 128), f32)

    w0t = jnp.zeros((128, F0), f32).at[:19, :500].set(W0.T)
    b0p = jnp.zeros((1, F0), f32).at[0, :500].set(b0)
    wct = _bf(jnp.zeros((F0, F0), f32).at[:500, :500].set(Wc.T))
    bcp = jnp.zeros((1, F0), f32).at[0, :500].set(bc)
    w1t = _bf(jnp.zeros((F0, F1), f32).at[:500, :300].set(W1.T))
    b1p = jnp.zeros((1, F1), f32).at[0, :300].set(b1)
    w2t = _bf(jnp.zeros((F1, F2), f32).at[:300, :100].set(W2.T))
    b2p = jnp.zeros((1, F2), f32).at[0, :100].set(b2)
    w3t = _bf(jnp.zeros((F2, 128), f32).at[:100, :6].set(W3.T))
    b3p = jnp.zeros((1, 128), f32).at[0, :6].set(b3)

    degp = _deg(dst32)
    degp = degp.reshape(32, NP, 1)
    hn4, dinv = _mlp0(degp, x_pad, w0t, b0p)
    agg_all = _agg(hn4.reshape(4 * NP, 128), src4, dst3, zrows)
    gout = _mlp2(agg_all.reshape(4, NP, 128), hn4, dinv,
                 wct, bcp, w1t, b1p, w2t, b2p, w3t, b3p, batch2d)
    return gout[:, :6]
